# Initial kernel scaffold; baseline (speedup 1.0000x reference)
import functools

import jax
import jax.numpy as jnp
from jax import lax
from jax.experimental import pallas as pl
from jax.experimental.pallas import tpu as pltpu
from jax.experimental.pallas import tpu_sc as plsc

N_NODES = 50000
H = 4
F = 32
IN = 128
BLK = 256
ROWS = 50176
ACC_ROWS = 51200
E_PAD = 262144
E_ROWS = E_PAD // 128


def _prep_body(feat_ref, w_ref, al_ref, ar_ref, heads_ref, el_ref, er_ref):
    i = pl.program_id(0)
    x = feat_ref[...]
    y = jnp.dot(x, w_ref[...], preferred_element_type=jnp.float32)
    row = i * BLK + lax.broadcasted_iota(jnp.int32, (BLK, 1), 0)
    y = jnp.where(row < N_NODES, y, 0.0)
    f_idx = lax.broadcasted_iota(jnp.int32, (IN, 16), 0)
    c_idx = lax.broadcasted_iota(jnp.int32, (IN, 16), 1)
    sel = ((f_idx // F) == c_idx).astype(jnp.float32)
    el_ref[...] = jnp.dot(y * al_ref[...], sel, preferred_element_type=jnp.float32)
    er_ref[...] = jnp.dot(y * ar_ref[...], sel, preferred_element_type=jnp.float32)
    for h in range(H):
        heads_ref[h] = y[:, h * F:(h + 1) * F]


def _tc_prep(feat, w_mat, al, ar):
    grid = ROWS // BLK
    return pl.pallas_call(
        _prep_body,
        grid=(grid,),
        in_specs=[pl.BlockSpec((BLK, IN), lambda i: (i, 0)),
                  pl.BlockSpec((IN, IN), lambda i: (0, 0)),
                  pl.BlockSpec((1, IN), lambda i: (0, 0)),
                  pl.BlockSpec((1, IN), lambda i: (0, 0))],
        out_specs=[pl.BlockSpec((H, BLK, F), lambda i: (0, i, 0)),
                   pl.BlockSpec((BLK, 16), lambda i: (i, 0)),
                   pl.BlockSpec((BLK, 16), lambda i: (i, 0))],
        out_shape=[jax.ShapeDtypeStruct((H, ROWS, F), jnp.float32),
                   jax.ShapeDtypeStruct((ROWS, 16), jnp.float32),
                   jax.ShapeDtypeStruct((ROWS, 16), jnp.float32)],
    )(feat, w_mat, al.reshape(1, IN), ar.reshape(1, IN))


def _sc_b1(el_tbl, er_tbl, src2, dst2):
    mesh = plsc.VectorSubcoreMesh(core_axis_name="c", subcore_axis_name="s")

    @functools.partial(
        pl.kernel,
        out_type=[jax.ShapeDtypeStruct((H, E_ROWS, 128), jnp.float32),
                  jax.ShapeDtypeStruct((2, ACC_ROWS, 16), jnp.float32)],
        mesh=mesh,
        scratch_types=[pltpu.VMEM((8, 128), jnp.int32),
                       pltpu.VMEM((8, 128), jnp.int32),
                       pltpu.VMEM((1024, 16), jnp.float32),
                       pltpu.VMEM((1024, 16), jnp.float32),
                       pltpu.VMEM((1024, 16), jnp.float32),
                       pltpu.VMEM((H, 8, 128), jnp.float32),
                       pltpu.VMEM((128, 16), jnp.float32),
                       pltpu.VMEM_SHARED((ACC_ROWS, 16), jnp.float32),
                       pltpu.SemaphoreType.DMA],
    )
    def k(el_hbm, er_hbm, src_hbm, dst_hbm, w_out, den_out,
          src_v, dst_v, el_r, er_r, w_v, wcols, zbuf, den_sh, sem):
        core = lax.axis_index("c")
        sub = lax.axis_index("s")
        gtid = core * 16 + sub

        @pl.loop(0, 128)
        def _z(i):
            zbuf[i] = jnp.zeros((16,), jnp.float32)

        @pl.loop(0, ACC_ROWS // (16 * 128))
        def _zc(j):
            pltpu.sync_copy(zbuf, den_sh.at[pl.ds(sub * (ACC_ROWS // 16) + j * 128, 128)])

        plsc.subcore_barrier()

        @pl.loop(0, 8)
        def _chunk(ci):
            rb = gtid * 64 + ci * 8
            pltpu.sync_copy(src_hbm.at[pl.ds(rb, 8)], src_v)
            pltpu.sync_copy(dst_hbm.at[pl.ds(rb, 8)], dst_v)
            cps = []
            for g in range(8):
                cps.append(pltpu.async_copy(
                    el_hbm.at[src_v.at[g]], el_r.at[pl.ds(g * 128, 128)], sem))
                cps.append(pltpu.async_copy(
                    er_hbm.at[dst_v.at[g]], er_r.at[pl.ds(g * 128, 128)], sem))
            for c in cps:
                c.wait()

            @pl.loop(0, 1024)
            def _e(e):
                v = el_r[e] + er_r[e]
                v = jnp.where(v >= 0.0, v, v * jnp.float32(0.2))
                w_v[e] = jnp.exp(v)

            for g in range(8):
                pltpu.sync_copy(w_v.at[pl.ds(g * 128, 128)],
                                den_sh.at[dst_v.at[g]], add=True)

            lanes = lax.iota(jnp.int32, 16)

            @pl.loop(0, 64)
            def _t(r16):
                rows = r16 * 16 + lanes
                r = r16 // 8
                c0 = (r16 % 8) * 16
                for h in range(H):
                    col = jnp.full((16,), h, jnp.int32)
                    wcols[h, r, pl.ds(c0, 16)] = plsc.load_gather(w_v, [rows, col])

            for h in range(H):
                pltpu.sync_copy(wcols.at[h], w_out.at[h, pl.ds(rb, 8)])

        plsc.subcore_barrier()
        pltpu.sync_copy(den_sh.at[pl.ds(sub * (ACC_ROWS // 16), ACC_ROWS // 16)],
                        den_out.at[core, pl.ds(sub * (ACC_ROWS // 16), ACC_ROWS // 16)])

    return k(el_tbl, er_tbl, src2, dst2)


def _sc_b2(feat_flat, src2, dst2, w_hbm_in):
    mesh = plsc.VectorSubcoreMesh(core_axis_name="c", subcore_axis_name="s")

    @functools.partial(
        pl.kernel,
        out_type=jax.ShapeDtypeStruct((H, ACC_ROWS, F), jnp.float32),
        mesh=mesh,
        scratch_types=[pltpu.VMEM((8, 128), jnp.int32),
                       pltpu.VMEM((8, 128), jnp.int32),
                       pltpu.VMEM((8, 128), jnp.int32),
                       pltpu.VMEM((8, 128), jnp.float32),
                       pltpu.VMEM((1024, F), jnp.float32),
                       pltpu.VMEM((128, F), jnp.float32),
                       pltpu.VMEM_SHARED((ACC_ROWS, F), jnp.float32),
                       pltpu.SemaphoreType.DMA],
    )
    def k(feat_hbm, src_hbm, dst_hbm, w_hbm, acc_out,
          src_v, dst_v, gidx_v, w_v, rows_v, zbuf, acc_sh, sem):
        core = lax.axis_index("c")
        sub = lax.axis_index("s")

        @pl.loop(0, 128)
        def _z(i):
            zbuf[i] = jnp.zeros((F,), jnp.float32)

        def run_head(h):
            @pl.loop(0, ACC_ROWS // (16 * 128))
            def _zc(j):
                pltpu.sync_copy(
                    zbuf, acc_sh.at[pl.ds(sub * (ACC_ROWS // 16) + j * 128, 128)])

            plsc.subcore_barrier()

            @pl.loop(0, 16)
            def _chunk(ci):
                rb = sub * 128 + ci * 8
                pltpu.sync_copy(src_hbm.at[pl.ds(rb, 8)], src_v)
                pltpu.sync_copy(dst_hbm.at[pl.ds(rb, 8)], dst_v)
                pltpu.sync_copy(w_hbm.at[h, pl.ds(rb, 8)], w_v)

                @pl.loop(0, 8)
                def _g(g):
                    for kk in range(8):
                        gidx_v[g, pl.ds(kk * 16, 16)] = (
                            src_v[g, pl.ds(kk * 16, 16)] + h * ROWS)

                cps = []
                for g in range(8):
                    cps.append(pltpu.async_copy(
                        feat_hbm.at[gidx_v.at[g]],
                        rows_v.at[pl.ds(g * 128, 128)], sem))
                for c in cps:
                    c.wait()

                @pl.loop(0, 1024)
                def _e(e):
                    hi = jnp.full((16,), e // 128, jnp.int32)
                    lo = jnp.full((16,), e % 128, jnp.int32)
                    wv = plsc.load_gather(w_v, [hi, lo])
                    rows_v[e, pl.ds(0, 16)] = rows_v[e, pl.ds(0, 16)] * wv
                    rows_v[e, pl.ds(16, 16)] = rows_v[e, pl.ds(16, 16)] * wv

                for g in range(8):
                    pltpu.sync_copy(rows_v.at[pl.ds(g * 128, 128)],
                                    acc_sh.at[dst_v.at[g]], add=True)

            plsc.subcore_barrier()
            pltpu.sync_copy(
                acc_sh.at[pl.ds(sub * (ACC_ROWS // 16), ACC_ROWS // 16)],
                acc_out.at[h, pl.ds(sub * (ACC_ROWS // 16), ACC_ROWS // 16)])
            plsc.subcore_barrier()

        for cc in range(2):
            @pl.when(core == cc)
            def _():
                for hi in range(2):
                    run_head(2 * cc + hi)

    return k(feat_flat, src2, dst2, w_hbm_in)


def _fin_body(a1_ref, a2_ref, d1_ref, d2_ref, b_ref, o_ref):
    d1 = d1_ref[0] + d1_ref[1]
    d2 = d2_ref[0] + d2_ref[1]
    parts = []
    for h in range(H):
        n1 = a1_ref[h] / jnp.maximum(d1[:, h:h + 1], 1e-20)
        n2 = a2_ref[h] / jnp.maximum(d2[:, h:h + 1], 1e-20)
        parts.append(n1 + n2)
    o_ref[...] = jnp.concatenate(parts, axis=1) + b_ref[...]


def _tc_fin(acc1, acc2, den1, den2, bias_sum):
    grid = ROWS // BLK
    return pl.pallas_call(
        _fin_body,
        grid=(grid,),
        in_specs=[pl.BlockSpec((H, BLK, F), lambda i: (0, i, 0)),
                  pl.BlockSpec((H, BLK, F), lambda i: (0, i, 0)),
                  pl.BlockSpec((2, BLK, 16), lambda i: (0, i, 0)),
                  pl.BlockSpec((2, BLK, 16), lambda i: (0, i, 0)),
                  pl.BlockSpec((1, IN), lambda i: (0, 0))],
        out_specs=pl.BlockSpec((BLK, IN), lambda i: (i, 0)),
        out_shape=jax.ShapeDtypeStruct((N_NODES, IN), jnp.float32),
    )(acc1, acc2, den1, den2, bias_sum)


def _pad_edges(ei):
    pad = E_PAD - ei.shape[1]
    src = jnp.concatenate([ei[0], jnp.full((pad,), N_NODES, jnp.int32)])
    dst = jnp.concatenate([ei[1], jnp.full((pad,), N_NODES, jnp.int32)])
    return src.reshape(E_ROWS, 128), dst.reshape(E_ROWS, 128)


def kernel(feat_item, feat_t, edge_index_i2t, edge_index_t2t,
           W_i2t, attn_l_i2t, attn_r_i2t, bias_i2t,
           W_t2t, attn_l_t2t, attn_r_t2t, bias_t2t):
    heads_i2t, el_i2t, _ = _tc_prep(feat_item, W_i2t, attn_l_i2t, attn_r_i2t)
    _, _, er_i2t = _tc_prep(feat_t, W_i2t, attn_l_i2t, attn_r_i2t)
    heads_t2t, el_t2t, er_t2t = _tc_prep(feat_t, W_t2t, attn_l_t2t, attn_r_t2t)

    s1, d1 = _pad_edges(edge_index_i2t)
    s2, d2 = _pad_edges(edge_index_t2t)

    w1, den1 = _sc_b1(el_i2t, er_i2t, s1, d1)
    w2, den2 = _sc_b1(el_t2t, er_t2t, s2, d2)

    acc1 = _sc_b2(heads_i2t.reshape(H * ROWS, F), s1, d1, w1)
    acc2 = _sc_b2(heads_t2t.reshape(H * ROWS, F), s2, d2, w2)

    out = _tc_fin(acc1, acc2, den1, den2,
                  (bias_i2t + bias_t2t).reshape(1, IN))
    return out.reshape(N_NODES, H, F)


# speedup vs baseline: 24.6868x; 24.6868x over previous
import dataclasses
import functools

import jax
import jax.numpy as jnp
from jax import lax
from jax.experimental import pallas as pl
from jax.experimental.pallas import tpu as pltpu
from jax.experimental.pallas import tpu_sc as plsc

N_NODES = 50000
H = 4
F = 32
IN = 128
BLK = 256
ROWS = 50176
ACC_ROWS = ROWS
TPR = ACC_ROWS // 16
E_PAD = 262144
E_ROWS = E_PAD // 128

_SC_PARAMS = pltpu.CompilerParams(
    needs_layout_passes=False, use_tc_tiling_on_sc=False)


def _prep_body(feat_ref, w_ref, al_ref, ar_ref, heads_ref, el_ref, er_ref):
    i = pl.program_id(0)
    x = feat_ref[...]
    y = jnp.dot(x, w_ref[...], preferred_element_type=jnp.float32)
    row = i * BLK + lax.broadcasted_iota(jnp.int32, (BLK, 1), 0)
    y = jnp.where(row < N_NODES, y, 0.0)
    f_idx = lax.broadcasted_iota(jnp.int32, (IN, 16), 0)
    c_idx = lax.broadcasted_iota(jnp.int32, (IN, 16), 1)
    sel = ((f_idx // F) == c_idx).astype(jnp.float32)
    el_ref[...] = jnp.dot(y * al_ref[...], sel, preferred_element_type=jnp.float32)
    er_ref[...] = jnp.dot(y * ar_ref[...], sel, preferred_element_type=jnp.float32)
    for h in range(H):
        heads_ref[h] = y[:, h * F:(h + 1) * F]


def _tc_prep(feat, w_mat, al, ar):
    grid = ROWS // BLK
    return pl.pallas_call(
        _prep_body,
        grid=(grid,),
        in_specs=[pl.BlockSpec((BLK, IN), lambda i: (i, 0)),
                  pl.BlockSpec((IN, IN), lambda i: (0, 0)),
                  pl.BlockSpec((1, IN), lambda i: (0, 0)),
                  pl.BlockSpec((1, IN), lambda i: (0, 0))],
        out_specs=[pl.BlockSpec((H, BLK, F), lambda i: (0, i, 0)),
                   pl.BlockSpec((BLK, 16), lambda i: (i, 0)),
                   pl.BlockSpec((BLK, 16), lambda i: (i, 0))],
        out_shape=[jax.ShapeDtypeStruct((H, ROWS, F), jnp.float32),
                   jax.ShapeDtypeStruct((ROWS, 16), jnp.float32),
                   jax.ShapeDtypeStruct((ROWS, 16), jnp.float32)],
    )(feat, w_mat, al.reshape(1, IN), ar.reshape(1, IN))


def _sc_b1(el_tbl, er_tbl, src2, dst2):
    mesh = plsc.VectorSubcoreMesh(core_axis_name="c", subcore_axis_name="s")

    @functools.partial(
        pl.kernel,
        out_type=[jax.ShapeDtypeStruct((H, E_ROWS, 128), jnp.float32),
                  jax.ShapeDtypeStruct((2, ACC_ROWS, 16), jnp.float32)],
        mesh=mesh,
        scratch_types=[pltpu.VMEM((8, 128), jnp.int32),
                       pltpu.VMEM((8, 128), jnp.int32),
                       pltpu.VMEM((1024, 16), jnp.float32),
                       pltpu.VMEM((1024, 16), jnp.float32),
                       pltpu.VMEM((1024, 16), jnp.float32),
                       pltpu.VMEM((H, 8, 128), jnp.float32),
                       pltpu.VMEM((392, 16), jnp.float32),
                       pltpu.VMEM_SHARED((ACC_ROWS, 16), jnp.float32),
                       pltpu.SemaphoreType.DMA],
        compiler_params=_SC_PARAMS,
    )
    def k(el_hbm, er_hbm, src_hbm, dst_hbm, w_out, den_out,
          src_v, dst_v, el_r, er_r, w_v, wcols, zbuf, den_sh, sem):
        core = lax.axis_index("c")
        sub = lax.axis_index("s")
        gtid = core * 16 + sub

        @pl.loop(0, 392)
        def _z(i):
            zbuf[i] = jnp.zeros((16,), jnp.float32)

        @pl.loop(0, TPR // 392)
        def _zc(j):
            pltpu.sync_copy(zbuf, den_sh.at[pl.ds(sub * TPR + j * 392, 392)])

        plsc.subcore_barrier()

        @pl.loop(0, 8)
        def _chunk(ci):
            rb = gtid * 64 + ci * 8
            pltpu.sync_copy(src_hbm.at[pl.ds(rb, 8)], src_v)
            pltpu.sync_copy(dst_hbm.at[pl.ds(rb, 8)], dst_v)
            cps = []
            for g in range(8):
                cps.append(pltpu.async_copy(
                    el_hbm.at[src_v.at[g]], el_r.at[pl.ds(g * 128, 128)], sem))
                cps.append(pltpu.async_copy(
                    er_hbm.at[dst_v.at[g]], er_r.at[pl.ds(g * 128, 128)], sem))
            for c in cps:
                c.wait()

            @pl.loop(0, 1024)
            def _e(e):
                v = el_r[e] + er_r[e]
                v = jnp.where(v >= 0.0, v, v * jnp.float32(0.2))
                w_v[e] = jnp.exp(v)

            for g in range(8):
                pltpu.sync_copy(w_v.at[pl.ds(g * 128, 128)],
                                den_sh.at[dst_v.at[g]], add=True)

            lanes = lax.iota(jnp.int32, 16)

            @pl.loop(0, 64)
            def _t(r16):
                rows = r16 * 16 + lanes
                r = r16 // 8
                c0 = (r16 % 8) * 16
                for h in range(H):
                    col = jnp.full((16,), h, jnp.int32)
                    wcols[h, r, pl.ds(c0, 16)] = plsc.load_gather(w_v, [rows, col])

            for h in range(H):
                pltpu.sync_copy(wcols.at[h], w_out.at[h, pl.ds(rb, 8)])

        plsc.subcore_barrier()
        pltpu.sync_copy(den_sh.at[pl.ds(sub * TPR, TPR)],
                        den_out.at[core, pl.ds(sub * TPR, TPR)])

    return k(el_tbl, er_tbl, src2, dst2)


def _sc_b2(feat_flat, src2, dst2, w_hbm_in):
    mesh = plsc.VectorSubcoreMesh(core_axis_name="c", subcore_axis_name="s")

    @functools.partial(
        pl.kernel,
        out_type=jax.ShapeDtypeStruct((H, ACC_ROWS, F), jnp.float32),
        mesh=mesh,
        scratch_types=[pltpu.VMEM((4, 128), jnp.int32),
                       pltpu.VMEM((4, 128), jnp.int32),
                       pltpu.VMEM((4, 128), jnp.float32),
                       pltpu.VMEM((512, F), jnp.float32),
                       pltpu.VMEM_SHARED((ACC_ROWS, F), jnp.float32),
                       pltpu.SemaphoreType.DMA],
        compiler_params=_SC_PARAMS,
    )
    def k(feat_hbm, src_hbm, dst_hbm, w_hbm, acc_out,
          src_v, dst_v, w_v, rows_v, acc_sh, sem):
        core = lax.axis_index("c")
        sub = lax.axis_index("s")

        def run_head(h):
            @pl.loop(0, 392)
            def _z(i):
                rows_v[i, pl.ds(0, 16)] = jnp.zeros((16,), jnp.float32)
                rows_v[i, pl.ds(16, 16)] = jnp.zeros((16,), jnp.float32)

            @pl.loop(0, TPR // 392)
            def _zc(j):
                pltpu.sync_copy(rows_v.at[pl.ds(0, 392)],
                                acc_sh.at[pl.ds(sub * TPR + j * 392, 392)])

            plsc.subcore_barrier()

            @pl.loop(0, 32)
            def _chunk(ci):
                rb = sub * 128 + ci * 4
                pltpu.sync_copy(src_hbm.at[pl.ds(rb, 4)], src_v)
                pltpu.sync_copy(dst_hbm.at[pl.ds(rb, 4)], dst_v)
                pltpu.sync_copy(w_hbm.at[h, pl.ds(rb, 4)], w_v)

                @pl.loop(0, 4)
                def _g(g):
                    for kk in range(8):
                        src_v[g, pl.ds(kk * 16, 16)] = (
                            src_v[g, pl.ds(kk * 16, 16)] + h * ROWS)

                cps = []
                for g in range(4):
                    cps.append(pltpu.async_copy(
                        feat_hbm.at[src_v.at[g]],
                        rows_v.at[pl.ds(g * 128, 128)], sem))
                for c in cps:
                    c.wait()

                @pl.loop(0, 512)
                def _e(e):
                    hi = jnp.full((16,), e // 128, jnp.int32)
                    lo = jnp.full((16,), e % 128, jnp.int32)
                    wv = plsc.load_gather(w_v, [hi, lo])
                    rows_v[e, pl.ds(0, 16)] = rows_v[e, pl.ds(0, 16)] * wv
                    rows_v[e, pl.ds(16, 16)] = rows_v[e, pl.ds(16, 16)] * wv

                for g in range(4):
                    pltpu.sync_copy(rows_v.at[pl.ds(g * 128, 128)],
                                    acc_sh.at[dst_v.at[g]], add=True)

            plsc.subcore_barrier()
            pltpu.sync_copy(acc_sh.at[pl.ds(sub * TPR, TPR)],
                            acc_out.at[h, pl.ds(sub * TPR, TPR)])
            plsc.subcore_barrier()

        for cc in range(2):
            @pl.when(core == cc)
            def _():
                for hi in range(2):
                    run_head(2 * cc + hi)

    return k(feat_flat, src2, dst2, w_hbm_in)


def _fin_body(a1_ref, a2_ref, d1_ref, d2_ref, b_ref, o_ref):
    d1 = d1_ref[0] + d1_ref[1]
    d2 = d2_ref[0] + d2_ref[1]
    parts = []
    for h in range(H):
        n1 = a1_ref[h] / jnp.maximum(d1[:, h:h + 1], 1e-20)
        n2 = a2_ref[h] / jnp.maximum(d2[:, h:h + 1], 1e-20)
        parts.append(n1 + n2)
    o_ref[...] = jnp.concatenate(parts, axis=1) + b_ref[...]


def _tc_fin(acc1, acc2, den1, den2, bias_sum):
    grid = ROWS // BLK
    return pl.pallas_call(
        _fin_body,
        grid=(grid,),
        in_specs=[pl.BlockSpec((H, BLK, F), lambda i: (0, i, 0)),
                  pl.BlockSpec((H, BLK, F), lambda i: (0, i, 0)),
                  pl.BlockSpec((2, BLK, 16), lambda i: (0, i, 0)),
                  pl.BlockSpec((2, BLK, 16), lambda i: (0, i, 0)),
                  pl.BlockSpec((1, IN), lambda i: (0, 0))],
        out_specs=pl.BlockSpec((BLK, IN), lambda i: (i, 0)),
        out_shape=jax.ShapeDtypeStruct((N_NODES, IN), jnp.float32),
    )(acc1, acc2, den1, den2, bias_sum)


def _pad_edges(ei):
    pad = E_PAD - ei.shape[1]
    src = jnp.concatenate([ei[0], jnp.full((pad,), N_NODES, jnp.int32)])
    dst = jnp.concatenate([ei[1], jnp.full((pad,), N_NODES, jnp.int32)])
    return src.reshape(E_ROWS, 128), dst.reshape(E_ROWS, 128)


def kernel(feat_item, feat_t, edge_index_i2t, edge_index_t2t,
           W_i2t, attn_l_i2t, attn_r_i2t, bias_i2t,
           W_t2t, attn_l_t2t, attn_r_t2t, bias_t2t):
    heads_i2t, el_i2t, _ = _tc_prep(feat_item, W_i2t, attn_l_i2t, attn_r_i2t)
    _, _, er_i2t = _tc_prep(feat_t, W_i2t, attn_l_i2t, attn_r_i2t)
    heads_t2t, el_t2t, er_t2t = _tc_prep(feat_t, W_t2t, attn_l_t2t, attn_r_t2t)

    s1, d1 = _pad_edges(edge_index_i2t)
    s2, d2 = _pad_edges(edge_index_t2t)

    w1, den1 = _sc_b1(el_i2t, er_i2t, s1, d1)
    w2, den2 = _sc_b1(el_t2t, er_t2t, s2, d2)

    acc1 = _sc_b2(heads_i2t.reshape(H * ROWS, F), s1, d1, w1)
    acc2 = _sc_b2(heads_t2t.reshape(H * ROWS, F), s2, d2, w2)

    out = _tc_fin(acc1, acc2, den1, den2,
                  (bias_i2t + bias_t2t).reshape(1, IN))
    return out.reshape(N_NODES, H, F)
